# baseline (speedup 1.0000x reference)
import functools

import jax
import jax.numpy as jnp
from jax import lax
from jax.experimental import pallas as pl
from jax.experimental.pallas import tpu as pltpu
from jax.experimental.pallas import tpu_sc as plsc

_VOCAB = 1_000_000
_DIM = 64
_B = 4096
_L = 200
_N = _B * _L

_INFO = plsc.get_sparse_core_info()
_NW = _INFO.num_cores * _INFO.num_subcores
_CHUNK = 128
_PER_W = _N // _NW
_NCHUNK = _PER_W // _CHUNK


def _embed_kernel(table_hbm, idx_hbm, out_hbm, idx_v, rows_v, sem_g):
    wid = lax.axis_index("s") * _INFO.num_cores + lax.axis_index("c")
    base = wid * _PER_W

    pltpu.sync_copy(idx_hbm.at[wid], idx_v)

    def body(g, _):
        pltpu.async_copy(table_hbm.at[idx_v.at[g]], rows_v, sem_g).wait()
        pltpu.sync_copy(rows_v, out_hbm.at[pl.ds(base + g * _CHUNK, _CHUNK)])
        return 0

    lax.fori_loop(0, _NCHUNK, body, 0)


@jax.jit
def _embed(table, idx_flat):
    mesh = plsc.VectorSubcoreMesh(core_axis_name="c", subcore_axis_name="s")
    run = pl.kernel(
        _embed_kernel,
        out_type=jax.ShapeDtypeStruct((_N, _DIM), jnp.float32),
        mesh=mesh,
        scratch_types=[
            pltpu.VMEM((_NCHUNK, _CHUNK), jnp.int32),
            pltpu.VMEM((_CHUNK, _DIM), jnp.float32),
            pltpu.SemaphoreType.DMA,
        ],
        compiler_params=pltpu.CompilerParams(use_tc_tiling_on_sc=False),
    )
    return run(table, idx_flat)


def kernel(sent_words, embed_weight):
    idx = sent_words.astype(jnp.int32).reshape(_NW, _NCHUNK, _CHUNK)
    out = _embed(embed_weight, idx)
    return out.reshape(_B, _L, _DIM)


# speedup vs baseline: 1.1177x; 1.1177x over previous
import jax
import jax.numpy as jnp
from jax import lax
from jax.experimental import pallas as pl
from jax.experimental.pallas import tpu as pltpu
from jax.experimental.pallas import tpu_sc as plsc

_VOCAB = 1_000_000
_DIM = 64
_B = 4096
_L = 200
_N = _B * _L

_INFO = plsc.get_sparse_core_info()
_NW = _INFO.num_cores * _INFO.num_subcores
_CHUNK = 128
_PER_W = _N // _NW
_NCHUNK = _PER_W // _CHUNK
_NBUF = 4
_NROUND = _NCHUNK // _NBUF


def _embed_kernel(table_hbm, idx_hbm, out_hbm, idx_v, rows_v, sems_g, sems_o):
    wid = lax.axis_index("s") * _INFO.num_cores + lax.axis_index("c")
    base = wid * _PER_W

    pltpu.sync_copy(idx_hbm.at[wid], idx_v)

    def gather(g, b):
        pltpu.async_copy(table_hbm.at[idx_v.at[g]], rows_v.at[b], sems_g[b])

    def out_slice(g):
        return out_hbm.at[pl.ds(base + g * _CHUNK, _CHUNK)]

    def wait_gather(g, b):
        pltpu.make_async_copy(out_slice(g), rows_v.at[b], sems_g[b]).wait()

    def writeback(g, b):
        pltpu.async_copy(rows_v.at[b], out_slice(g), sems_o[b])

    def wait_writeback(g, b):
        pltpu.make_async_copy(rows_v.at[b], out_slice(g), sems_o[b]).wait()

    for b in range(_NBUF):
        gather(b, b)

    def round_body(r, _):
        for b in range(_NBUF):
            g = r * _NBUF + b
            wait_gather(g, b)
            writeback(g, b)
            wait_writeback(g, b)
            gather(g + _NBUF, b)
        return 0

    lax.fori_loop(0, _NROUND - 1, round_body, 0)

    for b in range(_NBUF):
        g = _NCHUNK - _NBUF + b
        wait_gather(g, b)
        writeback(g, b)
    for b in range(_NBUF):
        g = _NCHUNK - _NBUF + b
        wait_writeback(g, b)


@jax.jit
def _embed(table, idx):
    mesh = plsc.VectorSubcoreMesh(core_axis_name="c", subcore_axis_name="s")
    run = pl.kernel(
        _embed_kernel,
        out_type=jax.ShapeDtypeStruct((_N, _DIM), jnp.float32),
        mesh=mesh,
        scratch_types=[
            pltpu.VMEM((_NCHUNK, _CHUNK), jnp.int32),
            pltpu.VMEM((_NBUF, _CHUNK, _DIM), jnp.float32),
            [pltpu.SemaphoreType.DMA] * _NBUF,
            [pltpu.SemaphoreType.DMA] * _NBUF,
        ],
        compiler_params=pltpu.CompilerParams(use_tc_tiling_on_sc=False),
    )
    return run(table, idx)


def kernel(sent_words, embed_weight):
    idx = sent_words.astype(jnp.int32).reshape(_NW, _NCHUNK, _CHUNK)
    out = _embed(embed_weight, idx)
    return out.reshape(_B, _L, _DIM)
